# Initial kernel scaffold; baseline (speedup 1.0000x reference)
import jax
import jax.numpy as jnp
from jax.experimental import pallas as pl


def kernel(x, edge_index, eps, We1, be1, We2, be2, Wmu, bmu, Wvar, bvar, Wd1, bd1, Wd2, bd2, Wd3, bd3):
    raise NotImplementedError("write your pallas kernel here")


# speedup vs baseline: 6.2910x; 6.2910x over previous
import functools

import jax
import jax.numpy as jnp
from jax import lax
from jax.experimental import pallas as pl
from jax.experimental.pallas import tpu as pltpu
from jax.experimental.pallas import tpu_sc as plsc

N = 100000
E = 3200000
NC = 2
NS = 16
NW = NC * NS
EPW = E // NW
CG = 800
CS = 2000
NPT = N // NS

_f32 = jnp.float32
_mesh = plsc.VectorSubcoreMesh(core_axis_name="c", subcore_axis_name="s")


@functools.partial(
    pl.kernel,
    mesh=_mesh,
    compiler_params=pltpu.CompilerParams(use_tc_tiling_on_sc=False),
    out_type=[
        jax.ShapeDtypeStruct((E, 32), _f32),
        jax.ShapeDtypeStruct((E, 32), _f32),
    ],
    scratch_types=[
        pltpu.VMEM((CG,), jnp.int32),
        pltpu.VMEM((CG,), jnp.int32),
        pltpu.VMEM((CG, 32), _f32),
        pltpu.VMEM((CG, 32), _f32),
        pltpu.SemaphoreType.DMA,
        pltpu.SemaphoreType.DMA,
    ],
)
def _sc_gather(p_hbm, q_hbm, dst_hbm, src_hbm, pg_hbm, qg_hbm,
               idxd, idxs, bufp, bufq, sem1, sem2):
    wid = lax.axis_index("s") * NC + lax.axis_index("c")

    def body(i, carry):
        base = wid * EPW + i * CG
        pltpu.sync_copy(dst_hbm.at[pl.ds(base, CG)], idxd)
        pltpu.sync_copy(src_hbm.at[pl.ds(base, CG)], idxs)
        cp1 = pltpu.async_copy(p_hbm.at[idxd], bufp, sem1)
        cp2 = pltpu.async_copy(q_hbm.at[idxs], bufq, sem2)
        cp1.wait()
        cp2.wait()
        pltpu.sync_copy(bufp, pg_hbm.at[pl.ds(base, CG)])
        pltpu.sync_copy(bufq, qg_hbm.at[pl.ds(base, CG)])
        return carry

    lax.fori_loop(0, EPW // CG, body, 0)


@functools.partial(
    pl.kernel,
    mesh=_mesh,
    compiler_params=pltpu.CompilerParams(use_tc_tiling_on_sc=False),
    out_type=jax.ShapeDtypeStruct((NC, N, 8), _f32),
    scratch_types=[
        pltpu.VMEM((CS,), jnp.int32),
        pltpu.VMEM((CS, 8), _f32),
        pltpu.VMEM_SHARED((N, 8), _f32),
        pltpu.SemaphoreType.DMA,
    ],
)
def _sc_scatter(m_hbm, dst_hbm, zeros_hbm, out_hbm, idx, buf, acc, sem):
    cid = lax.axis_index("c")
    sid = lax.axis_index("s")
    wid = sid * NC + cid
    pltpu.sync_copy(zeros_hbm, acc.at[pl.ds(sid * NPT, NPT)])
    plsc.subcore_barrier()

    def body(i, carry):
        base = wid * EPW + i * CS
        pltpu.sync_copy(dst_hbm.at[pl.ds(base, CS)], idx)
        pltpu.sync_copy(m_hbm.at[pl.ds(base, CS)], buf)
        pltpu.sync_copy(buf, acc.at[idx], add=True)
        return carry

    lax.fori_loop(0, EPW // CS, body, 0)
    plsc.subcore_barrier()
    pltpu.sync_copy(acc.at[pl.ds(sid * NPT, NPT)],
                    out_hbm.at[cid, pl.ds(sid * NPT, NPT)])


def _pre_body(x_ref, wa_ref, wb_ref, b_ref, p_ref, q_ref):
    x = x_ref[...]
    wb = wb_ref[...]
    q = jnp.dot(x, wb, preferred_element_type=_f32)
    p = jnp.dot(x, wa_ref[...] - wb, preferred_element_type=_f32) + b_ref[...]
    p_ref[...] = p
    q_ref[...] = q


def _tc_pre(x, wa, wb, b1, d_in):
    bn = 4000
    return pl.pallas_call(
        _pre_body,
        grid=(N // bn,),
        in_specs=[
            pl.BlockSpec((bn, d_in), lambda i: (i, 0)),
            pl.BlockSpec((d_in, 32), lambda i: (0, 0)),
            pl.BlockSpec((d_in, 32), lambda i: (0, 0)),
            pl.BlockSpec((1, 32), lambda i: (0, 0)),
        ],
        out_specs=[
            pl.BlockSpec((bn, 32), lambda i: (i, 0)),
            pl.BlockSpec((bn, 32), lambda i: (i, 0)),
        ],
        out_shape=[
            jax.ShapeDtypeStruct((N, 32), _f32),
            jax.ShapeDtypeStruct((N, 32), _f32),
        ],
    )(x, wa, wb, b1)


def _edge_body(with_count, pg_ref, qg_ref, w2_ref, b2_ref, w48_ref, m_ref):
    h = jnp.maximum(pg_ref[...] + qg_ref[...], 0.0)
    h = jnp.maximum(jnp.dot(h, w2_ref[...], preferred_element_type=_f32)
                    + b2_ref[...], 0.0)
    m = jnp.dot(h, w48_ref[...], preferred_element_type=_f32)
    if with_count:
        col = lax.broadcasted_iota(jnp.int32, m.shape, 1)
        m = m + jnp.where(col == 4, 1.0, 0.0).astype(_f32)
    m_ref[...] = m


def _tc_edge_mlp(pg, qg, w2, b2, w48, with_count):
    be = 8000
    return pl.pallas_call(
        functools.partial(_edge_body, with_count),
        grid=(E // be,),
        in_specs=[
            pl.BlockSpec((be, 32), lambda i: (i, 0)),
            pl.BlockSpec((be, 32), lambda i: (i, 0)),
            pl.BlockSpec((32, 32), lambda i: (0, 0)),
            pl.BlockSpec((1, 32), lambda i: (0, 0)),
            pl.BlockSpec((32, 8), lambda i: (0, 0)),
        ],
        out_specs=pl.BlockSpec((be, 8), lambda i: (i, 0)),
        out_shape=jax.ShapeDtypeStruct((E, 8), _f32),
    )(pg, qg, w2, b2, w48)


def _head_body(a_ref, eps_ref, bmu_ref, bvar_ref, wda_ref, wdb_ref, bd1_ref,
               mu_ref, lv_ref, p2_ref, q2_ref, ig_ref):
    s = a_ref[0] + a_ref[1]
    cnt = s[:, 4:5]
    inv = 1.0 / jnp.maximum(cnt, 1.0)
    mu = s[:, 0:2] * inv + bmu_ref[...]
    lv = s[:, 2:4] * inv + bvar_ref[...]
    z = mu + eps_ref[...] * jnp.exp(0.5 * lv)
    wdb = wdb_ref[...]
    p2 = jnp.dot(z, wda_ref[...] - wdb, preferred_element_type=_f32) \
        + bd1_ref[...]
    q2 = jnp.dot(z, wdb, preferred_element_type=_f32)
    mu_ref[...] = mu
    lv_ref[...] = lv
    p2_ref[...] = p2
    q2_ref[...] = q2
    ig_ref[...] = jnp.concatenate([inv, cnt * inv], axis=1)


def _tc_head(a, eps, bmu, bvar, wda, wdb, bd1):
    bn = 2000
    return pl.pallas_call(
        _head_body,
        grid=(N // bn,),
        in_specs=[
            pl.BlockSpec((NC, bn, 8), lambda i: (0, i, 0)),
            pl.BlockSpec((bn, 2), lambda i: (i, 0)),
            pl.BlockSpec((1, 2), lambda i: (0, 0)),
            pl.BlockSpec((1, 2), lambda i: (0, 0)),
            pl.BlockSpec((2, 32), lambda i: (0, 0)),
            pl.BlockSpec((2, 32), lambda i: (0, 0)),
            pl.BlockSpec((1, 32), lambda i: (0, 0)),
        ],
        out_specs=[
            pl.BlockSpec((bn, 2), lambda i: (i, 0)),
            pl.BlockSpec((bn, 2), lambda i: (i, 0)),
            pl.BlockSpec((bn, 32), lambda i: (i, 0)),
            pl.BlockSpec((bn, 32), lambda i: (i, 0)),
            pl.BlockSpec((bn, 2), lambda i: (i, 0)),
        ],
        out_shape=[
            jax.ShapeDtypeStruct((N, 2), _f32),
            jax.ShapeDtypeStruct((N, 2), _f32),
            jax.ShapeDtypeStruct((N, 32), _f32),
            jax.ShapeDtypeStruct((N, 32), _f32),
            jax.ShapeDtypeStruct((N, 2), _f32),
        ],
    )(a, eps, bmu, bvar, wda, wdb, bd1)


def _out_body(b_ref, ig_ref, bd3_ref, out_ref):
    s = b_ref[0] + b_ref[1]
    ig = ig_ref[...]
    out_ref[...] = (s[:, 0:4] * ig[:, 0:1] + ig[:, 1:2] * bd3_ref[...])


def _tc_out(b, ig, bd3):
    bn = 4000
    return pl.pallas_call(
        _out_body,
        grid=(N // bn,),
        in_specs=[
            pl.BlockSpec((NC, bn, 8), lambda i: (0, i, 0)),
            pl.BlockSpec((bn, 2), lambda i: (i, 0)),
            pl.BlockSpec((1, 4), lambda i: (0, 0)),
        ],
        out_specs=pl.BlockSpec((bn, 4), lambda i: (i, 0)),
        out_shape=jax.ShapeDtypeStruct((N, 4), _f32),
    )(b, ig, bd3)


def kernel(x, edge_index, eps, We1, be1, We2, be2, Wmu, bmu, Wvar, bvar,
           Wd1, bd1, Wd2, bd2, Wd3, bd3):
    src = edge_index[0]
    dst = edge_index[1]
    zeros_tile = jnp.zeros((NPT, 8), _f32)

    p, q = _tc_pre(x, We1[:4], We1[4:], be1.reshape(1, 32), 4)
    pg, qg = _sc_gather(p, q, dst, src)
    w48 = jnp.concatenate([Wmu, Wvar, jnp.zeros((32, 4), _f32)], axis=1)
    m = _tc_edge_mlp(pg, qg, We2, be2.reshape(1, 32), w48, True)
    a = _sc_scatter(m, dst, zeros_tile)

    mu, lv, p2, q2, ig = _tc_head(
        a, eps, bmu.reshape(1, 2), bvar.reshape(1, 2),
        Wd1[:2], Wd1[2:], bd1.reshape(1, 32))

    pg2, qg2 = _sc_gather(p2, q2, dst, src)
    wd38 = jnp.concatenate([Wd3, jnp.zeros((32, 4), _f32)], axis=1)
    m2 = _tc_edge_mlp(pg2, qg2, Wd2, bd2.reshape(1, 32), wd38, False)
    b = _sc_scatter(m2, dst, zeros_tile)
    out = _tc_out(b, ig, bd3.reshape(1, 4))
    return (out, mu, lv)


# speedup vs baseline: 15.6597x; 2.4892x over previous
import functools

import jax
import jax.numpy as jnp
from jax import lax
from jax.experimental import pallas as pl
from jax.experimental.pallas import tpu as pltpu
from jax.experimental.pallas import tpu_sc as plsc

N = 100000
E = 3200000
NC = 2
NS = 16
NW = NC * NS
EPW = E // NW
CG = 800
CS = 800
NPT = N // NS

_f32 = jnp.float32
_mesh = plsc.VectorSubcoreMesh(core_axis_name="c", subcore_axis_name="s")


@functools.partial(
    pl.kernel,
    mesh=_mesh,
    compiler_params=pltpu.CompilerParams(use_tc_tiling_on_sc=False),
    out_type=[
        jax.ShapeDtypeStruct((E, 32), _f32),
        jax.ShapeDtypeStruct((E, 32), _f32),
    ],
    scratch_types=[
        pltpu.VMEM((CG,), jnp.int32),
        pltpu.VMEM((CG,), jnp.int32),
        pltpu.VMEM((CG, 32), _f32),
        pltpu.VMEM((CG, 32), _f32),
        pltpu.SemaphoreType.DMA,
        pltpu.SemaphoreType.DMA,
    ],
)
def _sc_gather(p_hbm, q_hbm, dst_hbm, src_hbm, pg_hbm, qg_hbm,
               idxd, idxs, bufp, bufq, sem1, sem2):
    wid = lax.axis_index("s") * NC + lax.axis_index("c")

    def body(i, carry):
        base = wid * EPW + i * CG
        pltpu.sync_copy(dst_hbm.at[pl.ds(base, CG)], idxd)
        pltpu.sync_copy(src_hbm.at[pl.ds(base, CG)], idxs)
        cp1 = pltpu.async_copy(p_hbm.at[idxd], bufp, sem1)
        cp2 = pltpu.async_copy(q_hbm.at[idxs], bufq, sem2)
        cp1.wait()
        cp2.wait()
        pltpu.sync_copy(bufp, pg_hbm.at[pl.ds(base, CG)])
        pltpu.sync_copy(bufq, qg_hbm.at[pl.ds(base, CG)])
        return carry

    lax.fori_loop(0, EPW // CG, body, 0)


NH = N // NC
NACC = NH + NS
NPT2 = NH // NS
EPT = E // NS


@functools.partial(
    pl.kernel,
    mesh=_mesh,
    compiler_params=pltpu.CompilerParams(use_tc_tiling_on_sc=False),
    out_type=jax.ShapeDtypeStruct((NC, NH, 32), _f32),
    scratch_types=[
        pltpu.VMEM((CS,), jnp.int32),
        pltpu.VMEM((CS, 32), _f32),
        pltpu.VMEM_SHARED((NACC, 32), _f32),
        pltpu.SemaphoreType.DMA,
    ],
)
def _sc_scatter(m_hbm, dst_hbm, zeros_hbm, out_hbm, idx, buf, acc, sem):
    cid = lax.axis_index("c")
    sid = lax.axis_index("s")
    pltpu.sync_copy(zeros_hbm, acc.at[pl.ds(sid * (NACC // NS), NACC // NS)])
    plsc.subcore_barrier()
    lo = cid * NH
    trash = NH + sid

    def body(i, carry):
        base = sid * EPT + i * CS
        pltpu.sync_copy(dst_hbm.at[pl.ds(base, CS)], idx)
        pltpu.sync_copy(m_hbm.at[pl.ds(base, CS)], buf)

        def fix(v, carry2):
            iv = idx[pl.ds(v * 16, 16)]
            loc = iv - lo
            oob = (loc < 0) | (loc >= NH)
            idx[pl.ds(v * 16, 16)] = jnp.where(oob, trash, loc)
            return carry2

        lax.fori_loop(0, CS // 16, fix, 0)
        pltpu.sync_copy(buf, acc.at[idx], add=True)
        return carry

    lax.fori_loop(0, EPT // CS, body, 0)
    plsc.subcore_barrier()
    pltpu.sync_copy(acc.at[pl.ds(sid * NPT2, NPT2)],
                    out_hbm.at[cid, pl.ds(sid * NPT2, NPT2)])


def _pre_body(x_ref, wa_ref, wb_ref, b_ref, p_ref, q_ref):
    x = x_ref[...]
    wb = wb_ref[...]
    q = jnp.dot(x, wb, preferred_element_type=_f32)
    p = jnp.dot(x, wa_ref[...] - wb, preferred_element_type=_f32) + b_ref[...]
    p_ref[...] = p
    q_ref[...] = q


def _tc_pre(x, wa, wb, b1, d_in):
    bn = 4000
    return pl.pallas_call(
        _pre_body,
        grid=(N // bn,),
        in_specs=[
            pl.BlockSpec((bn, d_in), lambda i: (i, 0)),
            pl.BlockSpec((d_in, 32), lambda i: (0, 0)),
            pl.BlockSpec((d_in, 32), lambda i: (0, 0)),
            pl.BlockSpec((1, 32), lambda i: (0, 0)),
        ],
        out_specs=[
            pl.BlockSpec((bn, 32), lambda i: (i, 0)),
            pl.BlockSpec((bn, 32), lambda i: (i, 0)),
        ],
        out_shape=[
            jax.ShapeDtypeStruct((N, 32), _f32),
            jax.ShapeDtypeStruct((N, 32), _f32),
        ],
    )(x, wa, wb, b1)


def _edge_body(with_count, pg_ref, qg_ref, w2_ref, b2_ref, w48_ref, m_ref):
    h = jnp.maximum(pg_ref[...] + qg_ref[...], 0.0)
    h = jnp.maximum(jnp.dot(h, w2_ref[...], preferred_element_type=_f32)
                    + b2_ref[...], 0.0)
    m = jnp.dot(h, w48_ref[...], preferred_element_type=_f32)
    if with_count:
        col = lax.broadcasted_iota(jnp.int32, m.shape, 1)
        m = m + jnp.where(col % 32 == 4, 1.0, 0.0).astype(_f32)
    m_ref[...] = m


def _tc_edge_mlp(pg4, qg4, w2bd, b2t, w48bd, with_count):
    be4 = 4000
    return pl.pallas_call(
        functools.partial(_edge_body, with_count),
        grid=(E // (4 * be4),),
        in_specs=[
            pl.BlockSpec((be4, 128), lambda i: (i, 0)),
            pl.BlockSpec((be4, 128), lambda i: (i, 0)),
            pl.BlockSpec((128, 128), lambda i: (0, 0)),
            pl.BlockSpec((1, 128), lambda i: (0, 0)),
            pl.BlockSpec((128, 128), lambda i: (0, 0)),
        ],
        out_specs=pl.BlockSpec((be4, 128), lambda i: (i, 0)),
        out_shape=jax.ShapeDtypeStruct((E // 4, 128), _f32),
    )(pg4, qg4, w2bd, b2t, w48bd)


def _head_body(a_ref, eps_ref, bmu_ref, bvar_ref, wda_ref, wdb_ref, bd1_ref,
               mu_ref, lv_ref, p2_ref, q2_ref, ig_ref):
    s = a_ref[...]
    cnt = s[:, 4:5]
    inv = 1.0 / jnp.maximum(cnt, 1.0)
    mu = s[:, 0:2] * inv + bmu_ref[...]
    lv = s[:, 2:4] * inv + bvar_ref[...]
    z = mu + eps_ref[...] * jnp.exp(0.5 * lv)
    wdb = wdb_ref[...]
    p2 = jnp.dot(z, wda_ref[...] - wdb, preferred_element_type=_f32) \
        + bd1_ref[...]
    q2 = jnp.dot(z, wdb, preferred_element_type=_f32)
    mu_ref[...] = mu
    lv_ref[...] = lv
    p2_ref[...] = p2
    q2_ref[...] = q2
    ig_ref[...] = jnp.concatenate([inv, cnt * inv], axis=1)


def _tc_head(a, eps, bmu, bvar, wda, wdb, bd1):
    bn = 2000
    return pl.pallas_call(
        _head_body,
        grid=(N // bn,),
        in_specs=[
            pl.BlockSpec((bn, 32), lambda i: (i, 0)),
            pl.BlockSpec((bn, 2), lambda i: (i, 0)),
            pl.BlockSpec((1, 2), lambda i: (0, 0)),
            pl.BlockSpec((1, 2), lambda i: (0, 0)),
            pl.BlockSpec((2, 32), lambda i: (0, 0)),
            pl.BlockSpec((2, 32), lambda i: (0, 0)),
            pl.BlockSpec((1, 32), lambda i: (0, 0)),
        ],
        out_specs=[
            pl.BlockSpec((bn, 2), lambda i: (i, 0)),
            pl.BlockSpec((bn, 2), lambda i: (i, 0)),
            pl.BlockSpec((bn, 32), lambda i: (i, 0)),
            pl.BlockSpec((bn, 32), lambda i: (i, 0)),
            pl.BlockSpec((bn, 2), lambda i: (i, 0)),
        ],
        out_shape=[
            jax.ShapeDtypeStruct((N, 2), _f32),
            jax.ShapeDtypeStruct((N, 2), _f32),
            jax.ShapeDtypeStruct((N, 32), _f32),
            jax.ShapeDtypeStruct((N, 32), _f32),
            jax.ShapeDtypeStruct((N, 2), _f32),
        ],
    )(a, eps, bmu, bvar, wda, wdb, bd1)


def _out_body(b_ref, ig_ref, bd3_ref, out_ref):
    s = b_ref[...]
    ig = ig_ref[...]
    out_ref[...] = (s[:, 0:4] * ig[:, 0:1] + ig[:, 1:2] * bd3_ref[...])


def _tc_out(b, ig, bd3):
    bn = 4000
    return pl.pallas_call(
        _out_body,
        grid=(N // bn,),
        in_specs=[
            pl.BlockSpec((bn, 32), lambda i: (i, 0)),
            pl.BlockSpec((bn, 2), lambda i: (i, 0)),
            pl.BlockSpec((1, 4), lambda i: (0, 0)),
        ],
        out_specs=pl.BlockSpec((bn, 4), lambda i: (i, 0)),
        out_shape=jax.ShapeDtypeStruct((N, 4), _f32),
    )(b, ig, bd3)


def kernel(x, edge_index, eps, We1, be1, We2, be2, Wmu, bmu, Wvar, bvar,
           Wd1, bd1, Wd2, bd2, Wd3, bd3):
    src = edge_index[0]
    dst = edge_index[1]
    zeros_tile = jnp.zeros((NACC // NS, 32), _f32)
    eye4 = jnp.eye(4, dtype=_f32)

    p, q = _tc_pre(x, We1[:4], We1[4:], be1.reshape(1, 32), 4)
    pg, qg = _sc_gather(p, q, dst, src)
    w48 = jnp.concatenate([Wmu, Wvar, jnp.zeros((32, 28), _f32)], axis=1)
    m = _tc_edge_mlp(pg.reshape(E // 4, 128), qg.reshape(E // 4, 128),
                     jnp.kron(eye4, We2), jnp.tile(be2, 4).reshape(1, 128),
                     jnp.kron(eye4, w48), True)
    a = _sc_scatter(m.reshape(E, 32), dst, zeros_tile)

    mu, lv, p2, q2, ig = _tc_head(
        a.reshape(N, 32), eps, bmu.reshape(1, 2), bvar.reshape(1, 2),
        Wd1[:2], Wd1[2:], bd1.reshape(1, 32))

    pg2, qg2 = _sc_gather(p2, q2, dst, src)
    wd38 = jnp.concatenate([Wd3, jnp.zeros((32, 28), _f32)], axis=1)
    m2 = _tc_edge_mlp(pg2.reshape(E // 4, 128), qg2.reshape(E // 4, 128),
                      jnp.kron(eye4, Wd2), jnp.tile(bd2, 4).reshape(1, 128),
                      jnp.kron(eye4, wd38), False)
    b = _sc_scatter(m2.reshape(E, 32), dst, zeros_tile)
    out = _tc_out(b.reshape(N, 32), ig, bd3.reshape(1, 4))
    return (out, mu, lv)


# speedup vs baseline: 18.9407x; 1.2095x over previous
import functools

import jax
import jax.numpy as jnp
from jax import lax
from jax.experimental import pallas as pl
from jax.experimental.pallas import tpu as pltpu
from jax.experimental.pallas import tpu_sc as plsc

N = 100000
E = 3200000
NC = 2
NS = 16
NW = NC * NS
EPW = E // NW
CG = 400
CS = 400
NPT = N // NS

_f32 = jnp.float32
_mesh = plsc.VectorSubcoreMesh(core_axis_name="c", subcore_axis_name="s")


@functools.partial(
    pl.kernel,
    mesh=_mesh,
    compiler_params=pltpu.CompilerParams(use_tc_tiling_on_sc=False),
    out_type=[
        jax.ShapeDtypeStruct((E, 32), _f32),
        jax.ShapeDtypeStruct((E, 32), _f32),
    ],
    scratch_types=[
        pltpu.VMEM((CG,), jnp.int32),
        pltpu.VMEM((CG,), jnp.int32),
        pltpu.VMEM((CG,), jnp.int32),
        pltpu.VMEM((CG,), jnp.int32),
        pltpu.VMEM((CG, 32), _f32),
        pltpu.VMEM((CG, 32), _f32),
        pltpu.VMEM((CG, 32), _f32),
        pltpu.VMEM((CG, 32), _f32),
        pltpu.SemaphoreType.DMA,
        pltpu.SemaphoreType.DMA,
        pltpu.SemaphoreType.DMA,
        pltpu.SemaphoreType.DMA,
        pltpu.SemaphoreType.DMA,
        pltpu.SemaphoreType.DMA,
    ],
)
def _sc_gather(p_hbm, q_hbm, dst_hbm, src_hbm, pg_hbm, qg_hbm,
               idxd0, idxd1, idxs0, idxs1, bp0, bp1, bq0, bq1,
               si0, si1, sg0, sg1, so0, so1):
    wid = lax.axis_index("s") * NC + lax.axis_index("c")
    idxd, idxs = (idxd0, idxd1), (idxs0, idxs1)
    bp, bq = (bp0, bp1), (bq0, bq1)
    si, sg, so = (si0, si1), (sg0, sg1), (so0, so1)
    ni = EPW // CG

    def eb(i):
        return wid * EPW + i * CG

    def issue_idx(i, b):
        pltpu.async_copy(dst_hbm.at[pl.ds(eb(i), CG)], idxd[b], si[b])
        pltpu.async_copy(src_hbm.at[pl.ds(eb(i), CG)], idxs[b], si[b])

    def wait_idx(i, b):
        pltpu.make_async_copy(dst_hbm.at[pl.ds(eb(i), CG)], idxd[b], si[b]).wait()
        pltpu.make_async_copy(src_hbm.at[pl.ds(eb(i), CG)], idxs[b], si[b]).wait()

    def issue_gather(b):
        pltpu.async_copy(p_hbm.at[idxd[b]], bp[b], sg[b])
        pltpu.async_copy(q_hbm.at[idxs[b]], bq[b], sg[b])

    def wait_gather(b):
        pltpu.make_async_copy(p_hbm.at[idxd[b]], bp[b], sg[b]).wait()
        pltpu.make_async_copy(q_hbm.at[idxs[b]], bq[b], sg[b]).wait()

    def issue_store(i, b):
        pltpu.async_copy(bp[b], pg_hbm.at[pl.ds(eb(i), CG)], so[b])
        pltpu.async_copy(bq[b], qg_hbm.at[pl.ds(eb(i), CG)], so[b])

    def wait_store(i, b):
        pltpu.make_async_copy(bp[b], pg_hbm.at[pl.ds(eb(i), CG)], so[b]).wait()
        pltpu.make_async_copy(bq[b], qg_hbm.at[pl.ds(eb(i), CG)], so[b]).wait()

    issue_idx(0, 0)

    def body(i2, carry):
        for b in (0, 1):
            i = 2 * i2 + b
            o = 1 - b

            @pl.when(i >= 2)
            def _():
                wait_store(i - 2, b)

            @pl.when(i >= 1)
            def _():
                wait_gather(o)
                issue_store(i - 1, o)

            wait_idx(i, b)
            issue_gather(b)

            @pl.when(i + 1 < ni)
            def _():
                issue_idx(i + 1, o)
        return carry

    lax.fori_loop(0, ni // 2, body, 0)
    wait_gather(1)
    issue_store(ni - 1, 1)
    wait_store(ni - 2, 0)
    wait_store(ni - 1, 1)


NH = N // NC
NACC = NH + NS
NPT2 = NH // NS
EPT = E // NS


@functools.partial(
    pl.kernel,
    mesh=_mesh,
    compiler_params=pltpu.CompilerParams(use_tc_tiling_on_sc=False),
    out_type=jax.ShapeDtypeStruct((NC, NH, 32), _f32),
    scratch_types=[
        pltpu.VMEM((CS,), jnp.int32),
        pltpu.VMEM((CS,), jnp.int32),
        pltpu.VMEM((CS, 32), _f32),
        pltpu.VMEM((CS, 32), _f32),
        pltpu.VMEM_SHARED((NACC, 32), _f32),
        pltpu.SemaphoreType.DMA,
        pltpu.SemaphoreType.DMA,
        pltpu.SemaphoreType.DMA,
        pltpu.SemaphoreType.DMA,
    ],
)
def _sc_scatter(m_hbm, dst_hbm, zeros_hbm, out_hbm,
                idx0, idx1, buf0, buf1, acc, sl0, sl1, ss0, ss1):
    cid = lax.axis_index("c")
    sid = lax.axis_index("s")
    pltpu.sync_copy(zeros_hbm, acc.at[pl.ds(sid * (NACC // NS), NACC // NS)])
    plsc.subcore_barrier()
    lo = cid * NH
    trash = NH + sid
    idx, buf = (idx0, idx1), (buf0, buf1)
    sl, ss = (sl0, sl1), (ss0, ss1)
    ni = EPT // CS

    def eb(i):
        return sid * EPT + i * CS

    def issue_load(i, b):
        pltpu.async_copy(dst_hbm.at[pl.ds(eb(i), CS)], idx[b], sl[b])
        pltpu.async_copy(m_hbm.at[pl.ds(eb(i), CS)], buf[b], sl[b])

    def wait_load(i, b):
        pltpu.make_async_copy(dst_hbm.at[pl.ds(eb(i), CS)], idx[b], sl[b]).wait()
        pltpu.make_async_copy(m_hbm.at[pl.ds(eb(i), CS)], buf[b], sl[b]).wait()

    def wait_scat(b):
        pltpu.make_async_copy(buf[b], acc.at[idx[b]], ss[b]).wait()

    issue_load(0, 0)

    def body(i2, carry):
        for b in (0, 1):
            i = 2 * i2 + b
            o = 1 - b
            wait_load(i, b)

            def fix(v, carry2):
                iv = idx[b][pl.ds(v * 16, 16)]
                loc = iv - lo
                oob = (loc < 0) | (loc >= NH)
                idx[b][pl.ds(v * 16, 16)] = jnp.where(oob, trash, loc)
                return carry2

            lax.fori_loop(0, CS // 16, fix, 0)
            pltpu.async_copy(buf[b], acc.at[idx[b]], ss[b], add=True)

            @pl.when(i + 1 < ni)
            def _():
                @pl.when(i >= 1)
                def _():
                    wait_scat(o)

                issue_load(i + 1, o)
        return carry

    lax.fori_loop(0, ni // 2, body, 0)
    wait_scat(0)
    wait_scat(1)
    plsc.subcore_barrier()
    pltpu.sync_copy(acc.at[pl.ds(sid * NPT2, NPT2)],
                    out_hbm.at[cid, pl.ds(sid * NPT2, NPT2)])


def _pre_body(x_ref, wa_ref, wb_ref, b_ref, p_ref, q_ref):
    x = x_ref[...]
    wb = wb_ref[...]
    q = jnp.dot(x, wb, preferred_element_type=_f32)
    p = jnp.dot(x, wa_ref[...] - wb, preferred_element_type=_f32) + b_ref[...]
    p_ref[...] = p
    q_ref[...] = q


def _tc_pre(x, wa, wb, b1, d_in):
    bn = 4000
    return pl.pallas_call(
        _pre_body,
        grid=(N // bn,),
        in_specs=[
            pl.BlockSpec((bn, d_in), lambda i: (i, 0)),
            pl.BlockSpec((d_in, 32), lambda i: (0, 0)),
            pl.BlockSpec((d_in, 32), lambda i: (0, 0)),
            pl.BlockSpec((1, 32), lambda i: (0, 0)),
        ],
        out_specs=[
            pl.BlockSpec((bn, 32), lambda i: (i, 0)),
            pl.BlockSpec((bn, 32), lambda i: (i, 0)),
        ],
        out_shape=[
            jax.ShapeDtypeStruct((N, 32), _f32),
            jax.ShapeDtypeStruct((N, 32), _f32),
        ],
    )(x, wa, wb, b1)


def _edge_body(with_count, pg_ref, qg_ref, w2_ref, b2_ref, w48_ref, m_ref):
    h = jnp.maximum(pg_ref[...] + qg_ref[...], 0.0)
    h = jnp.maximum(jnp.dot(h, w2_ref[...], preferred_element_type=_f32)
                    + b2_ref[...], 0.0)
    m = jnp.dot(h, w48_ref[...], preferred_element_type=_f32)
    if with_count:
        col = lax.broadcasted_iota(jnp.int32, m.shape, 1)
        m = m + jnp.where(col % 32 == 4, 1.0, 0.0).astype(_f32)
    m_ref[...] = m


def _tc_edge_mlp(pg4, qg4, w2bd, b2t, w48bd, with_count):
    be4 = 4000
    return pl.pallas_call(
        functools.partial(_edge_body, with_count),
        grid=(E // (4 * be4),),
        in_specs=[
            pl.BlockSpec((be4, 128), lambda i: (i, 0)),
            pl.BlockSpec((be4, 128), lambda i: (i, 0)),
            pl.BlockSpec((128, 128), lambda i: (0, 0)),
            pl.BlockSpec((1, 128), lambda i: (0, 0)),
            pl.BlockSpec((128, 128), lambda i: (0, 0)),
        ],
        out_specs=pl.BlockSpec((be4, 128), lambda i: (i, 0)),
        out_shape=jax.ShapeDtypeStruct((E // 4, 128), _f32),
    )(pg4, qg4, w2bd, b2t, w48bd)


def _head_body(a_ref, eps_ref, bmu_ref, bvar_ref, wda_ref, wdb_ref, bd1_ref,
               mu_ref, lv_ref, p2_ref, q2_ref, ig_ref):
    s = a_ref[...]
    cnt = s[:, 4:5]
    inv = 1.0 / jnp.maximum(cnt, 1.0)
    mu = s[:, 0:2] * inv + bmu_ref[...]
    lv = s[:, 2:4] * inv + bvar_ref[...]
    z = mu + eps_ref[...] * jnp.exp(0.5 * lv)
    wdb = wdb_ref[...]
    p2 = jnp.dot(z, wda_ref[...] - wdb, preferred_element_type=_f32) \
        + bd1_ref[...]
    q2 = jnp.dot(z, wdb, preferred_element_type=_f32)
    mu_ref[...] = mu
    lv_ref[...] = lv
    p2_ref[...] = p2
    q2_ref[...] = q2
    ig_ref[...] = jnp.concatenate([inv, cnt * inv], axis=1)


def _tc_head(a, eps, bmu, bvar, wda, wdb, bd1):
    bn = 2000
    return pl.pallas_call(
        _head_body,
        grid=(N // bn,),
        in_specs=[
            pl.BlockSpec((bn, 32), lambda i: (i, 0)),
            pl.BlockSpec((bn, 2), lambda i: (i, 0)),
            pl.BlockSpec((1, 2), lambda i: (0, 0)),
            pl.BlockSpec((1, 2), lambda i: (0, 0)),
            pl.BlockSpec((2, 32), lambda i: (0, 0)),
            pl.BlockSpec((2, 32), lambda i: (0, 0)),
            pl.BlockSpec((1, 32), lambda i: (0, 0)),
        ],
        out_specs=[
            pl.BlockSpec((bn, 2), lambda i: (i, 0)),
            pl.BlockSpec((bn, 2), lambda i: (i, 0)),
            pl.BlockSpec((bn, 32), lambda i: (i, 0)),
            pl.BlockSpec((bn, 32), lambda i: (i, 0)),
            pl.BlockSpec((bn, 2), lambda i: (i, 0)),
        ],
        out_shape=[
            jax.ShapeDtypeStruct((N, 2), _f32),
            jax.ShapeDtypeStruct((N, 2), _f32),
            jax.ShapeDtypeStruct((N, 32), _f32),
            jax.ShapeDtypeStruct((N, 32), _f32),
            jax.ShapeDtypeStruct((N, 2), _f32),
        ],
    )(a, eps, bmu, bvar, wda, wdb, bd1)


def _out_body(b_ref, ig_ref, bd3_ref, out_ref):
    s = b_ref[...]
    ig = ig_ref[...]
    out_ref[...] = (s[:, 0:4] * ig[:, 0:1] + ig[:, 1:2] * bd3_ref[...])


def _tc_out(b, ig, bd3):
    bn = 4000
    return pl.pallas_call(
        _out_body,
        grid=(N // bn,),
        in_specs=[
            pl.BlockSpec((bn, 32), lambda i: (i, 0)),
            pl.BlockSpec((bn, 2), lambda i: (i, 0)),
            pl.BlockSpec((1, 4), lambda i: (0, 0)),
        ],
        out_specs=pl.BlockSpec((bn, 4), lambda i: (i, 0)),
        out_shape=jax.ShapeDtypeStruct((N, 4), _f32),
    )(b, ig, bd3)


def kernel(x, edge_index, eps, We1, be1, We2, be2, Wmu, bmu, Wvar, bvar,
           Wd1, bd1, Wd2, bd2, Wd3, bd3):
    src = edge_index[0]
    dst = edge_index[1]
    zeros_tile = jnp.zeros((NACC // NS, 32), _f32)
    eye4 = jnp.eye(4, dtype=_f32)

    p, q = _tc_pre(x, We1[:4], We1[4:], be1.reshape(1, 32), 4)
    pg, qg = _sc_gather(p, q, dst, src)
    w48 = jnp.concatenate([Wmu, Wvar, jnp.zeros((32, 28), _f32)], axis=1)
    m = _tc_edge_mlp(pg.reshape(E // 4, 128), qg.reshape(E // 4, 128),
                     jnp.kron(eye4, We2), jnp.tile(be2, 4).reshape(1, 128),
                     jnp.kron(eye4, w48), True)
    a = _sc_scatter(m.reshape(E, 32), dst, zeros_tile)

    mu, lv, p2, q2, ig = _tc_head(
        a.reshape(N, 32), eps, bmu.reshape(1, 2), bvar.reshape(1, 2),
        Wd1[:2], Wd1[2:], bd1.reshape(1, 32))

    pg2, qg2 = _sc_gather(p2, q2, dst, src)
    wd38 = jnp.concatenate([Wd3, jnp.zeros((32, 28), _f32)], axis=1)
    m2 = _tc_edge_mlp(pg2.reshape(E // 4, 128), qg2.reshape(E // 4, 128),
                      jnp.kron(eye4, Wd2), jnp.tile(bd2, 4).reshape(1, 128),
                      jnp.kron(eye4, wd38), False)
    b = _sc_scatter(m2.reshape(E, 32), dst, zeros_tile)
    out = _tc_out(b.reshape(N, 32), ig, bd3.reshape(1, 4))
    return (out, mu, lv)


# speedup vs baseline: 22.6351x; 1.1951x over previous
import functools

import jax
import jax.numpy as jnp
from jax import lax
from jax.experimental import pallas as pl
from jax.experimental.pallas import tpu as pltpu
from jax.experimental.pallas import tpu_sc as plsc

N = 100000
E = 3200000
NC = 2
NS = 16
NW = NC * NS
EPW = E // NW
CG = 400
CS = 2000
NPT = N // NS

_f32 = jnp.float32
_mesh = plsc.VectorSubcoreMesh(core_axis_name="c", subcore_axis_name="s")


@functools.partial(
    pl.kernel,
    mesh=_mesh,
    compiler_params=pltpu.CompilerParams(use_tc_tiling_on_sc=False),
    out_type=[
        jax.ShapeDtypeStruct((E, 32), _f32),
        jax.ShapeDtypeStruct((E, 32), _f32),
    ],
    scratch_types=[
        pltpu.VMEM((CG,), jnp.int32),
        pltpu.VMEM((CG,), jnp.int32),
        pltpu.VMEM((CG,), jnp.int32),
        pltpu.VMEM((CG,), jnp.int32),
        pltpu.VMEM((CG, 32), _f32),
        pltpu.VMEM((CG, 32), _f32),
        pltpu.VMEM((CG, 32), _f32),
        pltpu.VMEM((CG, 32), _f32),
        pltpu.SemaphoreType.DMA,
        pltpu.SemaphoreType.DMA,
        pltpu.SemaphoreType.DMA,
        pltpu.SemaphoreType.DMA,
        pltpu.SemaphoreType.DMA,
        pltpu.SemaphoreType.DMA,
    ],
)
def _sc_gather(p_hbm, q_hbm, dst_hbm, src_hbm, pg_hbm, qg_hbm,
               idxd0, idxd1, idxs0, idxs1, bp0, bp1, bq0, bq1,
               si0, si1, sg0, sg1, so0, so1):
    wid = lax.axis_index("s") * NC + lax.axis_index("c")
    idxd, idxs = (idxd0, idxd1), (idxs0, idxs1)
    bp, bq = (bp0, bp1), (bq0, bq1)
    si, sg, so = (si0, si1), (sg0, sg1), (so0, so1)
    ni = EPW // CG

    def eb(i):
        return wid * EPW + i * CG

    def issue_idx(i, b):
        pltpu.async_copy(dst_hbm.at[pl.ds(eb(i), CG)], idxd[b], si[b])
        pltpu.async_copy(src_hbm.at[pl.ds(eb(i), CG)], idxs[b], si[b])

    def wait_idx(i, b):
        pltpu.make_async_copy(dst_hbm.at[pl.ds(eb(i), CG)], idxd[b], si[b]).wait()
        pltpu.make_async_copy(src_hbm.at[pl.ds(eb(i), CG)], idxs[b], si[b]).wait()

    def issue_gather(b):
        pltpu.async_copy(p_hbm.at[idxd[b]], bp[b], sg[b])
        pltpu.async_copy(q_hbm.at[idxs[b]], bq[b], sg[b])

    def wait_gather(b):
        pltpu.make_async_copy(p_hbm.at[idxd[b]], bp[b], sg[b]).wait()
        pltpu.make_async_copy(q_hbm.at[idxs[b]], bq[b], sg[b]).wait()

    def issue_store(i, b):
        pltpu.async_copy(bp[b], pg_hbm.at[pl.ds(eb(i), CG)], so[b])
        pltpu.async_copy(bq[b], qg_hbm.at[pl.ds(eb(i), CG)], so[b])

    def wait_store(i, b):
        pltpu.make_async_copy(bp[b], pg_hbm.at[pl.ds(eb(i), CG)], so[b]).wait()
        pltpu.make_async_copy(bq[b], qg_hbm.at[pl.ds(eb(i), CG)], so[b]).wait()

    issue_idx(0, 0)

    def body(i2, carry):
        for b in (0, 1):
            i = 2 * i2 + b
            o = 1 - b

            @pl.when(i >= 2)
            def _():
                wait_store(i - 2, b)

            @pl.when(i >= 1)
            def _():
                wait_gather(o)
                issue_store(i - 1, o)

            wait_idx(i, b)
            issue_gather(b)

            @pl.when(i + 1 < ni)
            def _():
                issue_idx(i + 1, o)
        return carry

    lax.fori_loop(0, ni // 2, body, 0)
    wait_gather(1)
    issue_store(ni - 1, 1)
    wait_store(ni - 2, 0)
    wait_store(ni - 1, 1)


NH = N // NC
NACC = NH + NS
NPT2 = NH // NS
EPT = E // NS


@functools.partial(
    pl.kernel,
    mesh=_mesh,
    compiler_params=pltpu.CompilerParams(use_tc_tiling_on_sc=False),
    out_type=jax.ShapeDtypeStruct((NC, NH, 8), _f32),
    scratch_types=[
        pltpu.VMEM((CS,), jnp.int32),
        pltpu.VMEM((CS,), jnp.int32),
        pltpu.VMEM((CS, 8), _f32),
        pltpu.VMEM((CS, 8), _f32),
        pltpu.VMEM_SHARED((NACC, 8), _f32),
        pltpu.SemaphoreType.DMA,
        pltpu.SemaphoreType.DMA,
        pltpu.SemaphoreType.DMA,
        pltpu.SemaphoreType.DMA,
    ],
)
def _sc_scatter(m_hbm, dst_hbm, zeros_hbm, out_hbm,
                idx0, idx1, buf0, buf1, acc, sl0, sl1, ss0, ss1):
    cid = lax.axis_index("c")
    sid = lax.axis_index("s")
    pltpu.sync_copy(zeros_hbm, acc.at[pl.ds(sid * (NACC // NS), NACC // NS)])
    plsc.subcore_barrier()
    lo = cid * NH
    trash = NH + sid
    idx, buf = (idx0, idx1), (buf0, buf1)
    sl, ss = (sl0, sl1), (ss0, ss1)
    ni = EPT // CS

    def eb(i):
        return sid * EPT + i * CS

    def issue_load(i, b):
        pltpu.async_copy(dst_hbm.at[pl.ds(eb(i), CS)], idx[b], sl[b])
        pltpu.async_copy(m_hbm.at[pl.ds(eb(i), CS), pl.ds(0, 8)], buf[b], sl[b])

    def wait_load(i, b):
        pltpu.make_async_copy(dst_hbm.at[pl.ds(eb(i), CS)], idx[b], sl[b]).wait()
        pltpu.make_async_copy(m_hbm.at[pl.ds(eb(i), CS), pl.ds(0, 8)],
                              buf[b], sl[b]).wait()

    def wait_scat(b):
        pltpu.make_async_copy(buf[b], acc.at[idx[b]], ss[b]).wait()

    issue_load(0, 0)

    def body(i2, carry):
        for b in (0, 1):
            i = 2 * i2 + b
            o = 1 - b
            wait_load(i, b)

            def fix(v, carry2):
                iv = idx[b][pl.ds(v * 16, 16)]
                loc = iv - lo
                oob = (loc < 0) | (loc >= NH)
                idx[b][pl.ds(v * 16, 16)] = jnp.where(oob, trash, loc)
                return carry2

            lax.fori_loop(0, CS // 16, fix, 0)
            pltpu.async_copy(buf[b], acc.at[idx[b]], ss[b], add=True)

            @pl.when(i + 1 < ni)
            def _():
                @pl.when(i >= 1)
                def _():
                    wait_scat(o)

                issue_load(i + 1, o)
        return carry

    lax.fori_loop(0, ni // 2, body, 0)
    wait_scat(0)
    wait_scat(1)
    plsc.subcore_barrier()
    pltpu.sync_copy(acc.at[pl.ds(sid * NPT2, NPT2)],
                    out_hbm.at[cid, pl.ds(sid * NPT2, NPT2)])


def _pre_body(x_ref, wa_ref, wb_ref, b_ref, p_ref, q_ref):
    x = x_ref[...]
    wb = wb_ref[...]
    q = jnp.dot(x, wb, preferred_element_type=_f32)
    p = jnp.dot(x, wa_ref[...] - wb, preferred_element_type=_f32) + b_ref[...]
    p_ref[...] = p
    q_ref[...] = q


def _tc_pre(x, wa, wb, b1, d_in):
    bn = 4000
    return pl.pallas_call(
        _pre_body,
        grid=(N // bn,),
        in_specs=[
            pl.BlockSpec((bn, d_in), lambda i: (i, 0)),
            pl.BlockSpec((d_in, 32), lambda i: (0, 0)),
            pl.BlockSpec((d_in, 32), lambda i: (0, 0)),
            pl.BlockSpec((1, 32), lambda i: (0, 0)),
        ],
        out_specs=[
            pl.BlockSpec((bn, 32), lambda i: (i, 0)),
            pl.BlockSpec((bn, 32), lambda i: (i, 0)),
        ],
        out_shape=[
            jax.ShapeDtypeStruct((N, 32), _f32),
            jax.ShapeDtypeStruct((N, 32), _f32),
        ],
    )(x, wa, wb, b1)


def _edge_body(with_count, pg_ref, qg_ref, w2_ref, b2_ref, w48_ref, m_ref):
    h = jnp.maximum(pg_ref[...] + qg_ref[...], 0.0)
    h = jnp.maximum(jnp.dot(h, w2_ref[...], preferred_element_type=_f32)
                    + b2_ref[...], 0.0)
    m = jnp.dot(h, w48_ref[...], preferred_element_type=_f32)
    if with_count:
        col = lax.broadcasted_iota(jnp.int32, m.shape, 1)
        m = m + jnp.where(col % 32 == 4, 1.0, 0.0).astype(_f32)
    m_ref[...] = m


def _tc_edge_mlp(pg4, qg4, w2bd, b2t, w48bd, with_count):
    be4 = 4000
    return pl.pallas_call(
        functools.partial(_edge_body, with_count),
        grid=(E // (4 * be4),),
        in_specs=[
            pl.BlockSpec((be4, 128), lambda i: (i, 0)),
            pl.BlockSpec((be4, 128), lambda i: (i, 0)),
            pl.BlockSpec((128, 128), lambda i: (0, 0)),
            pl.BlockSpec((1, 128), lambda i: (0, 0)),
            pl.BlockSpec((128, 128), lambda i: (0, 0)),
        ],
        out_specs=pl.BlockSpec((be4, 128), lambda i: (i, 0)),
        out_shape=jax.ShapeDtypeStruct((E // 4, 128), _f32),
    )(pg4, qg4, w2bd, b2t, w48bd)


def _head_body(a_ref, eps_ref, bmu_ref, bvar_ref, wda_ref, wdb_ref, bd1_ref,
               mu_ref, lv_ref, p2_ref, q2_ref, ig_ref):
    s = a_ref[...]
    cnt = s[:, 4:5]
    inv = 1.0 / jnp.maximum(cnt, 1.0)
    mu = s[:, 0:2] * inv + bmu_ref[...]
    lv = s[:, 2:4] * inv + bvar_ref[...]
    z = mu + eps_ref[...] * jnp.exp(0.5 * lv)
    wdb = wdb_ref[...]
    p2 = jnp.dot(z, wda_ref[...] - wdb, preferred_element_type=_f32) \
        + bd1_ref[...]
    q2 = jnp.dot(z, wdb, preferred_element_type=_f32)
    mu_ref[...] = mu
    lv_ref[...] = lv
    p2_ref[...] = p2
    q2_ref[...] = q2
    ig_ref[...] = jnp.concatenate([inv, cnt * inv], axis=1)


def _tc_head(a, eps, bmu, bvar, wda, wdb, bd1):
    bn = 2000
    return pl.pallas_call(
        _head_body,
        grid=(N // bn,),
        in_specs=[
            pl.BlockSpec((bn, 8), lambda i: (i, 0)),
            pl.BlockSpec((bn, 2), lambda i: (i, 0)),
            pl.BlockSpec((1, 2), lambda i: (0, 0)),
            pl.BlockSpec((1, 2), lambda i: (0, 0)),
            pl.BlockSpec((2, 32), lambda i: (0, 0)),
            pl.BlockSpec((2, 32), lambda i: (0, 0)),
            pl.BlockSpec((1, 32), lambda i: (0, 0)),
        ],
        out_specs=[
            pl.BlockSpec((bn, 2), lambda i: (i, 0)),
            pl.BlockSpec((bn, 2), lambda i: (i, 0)),
            pl.BlockSpec((bn, 32), lambda i: (i, 0)),
            pl.BlockSpec((bn, 32), lambda i: (i, 0)),
            pl.BlockSpec((bn, 2), lambda i: (i, 0)),
        ],
        out_shape=[
            jax.ShapeDtypeStruct((N, 2), _f32),
            jax.ShapeDtypeStruct((N, 2), _f32),
            jax.ShapeDtypeStruct((N, 32), _f32),
            jax.ShapeDtypeStruct((N, 32), _f32),
            jax.ShapeDtypeStruct((N, 2), _f32),
        ],
    )(a, eps, bmu, bvar, wda, wdb, bd1)


def _out_body(b_ref, ig_ref, bd3_ref, out_ref):
    s = b_ref[...]
    ig = ig_ref[...]
    out_ref[...] = (s[:, 0:4] * ig[:, 0:1] + ig[:, 1:2] * bd3_ref[...])


def _tc_out(b, ig, bd3):
    bn = 4000
    return pl.pallas_call(
        _out_body,
        grid=(N // bn,),
        in_specs=[
            pl.BlockSpec((bn, 8), lambda i: (i, 0)),
            pl.BlockSpec((bn, 2), lambda i: (i, 0)),
            pl.BlockSpec((1, 4), lambda i: (0, 0)),
        ],
        out_specs=pl.BlockSpec((bn, 4), lambda i: (i, 0)),
        out_shape=jax.ShapeDtypeStruct((N, 4), _f32),
    )(b, ig, bd3)


def kernel(x, edge_index, eps, We1, be1, We2, be2, Wmu, bmu, Wvar, bvar,
           Wd1, bd1, Wd2, bd2, Wd3, bd3):
    src = edge_index[0]
    dst = edge_index[1]
    zeros_tile = jnp.zeros((NACC // NS, 8), _f32)
    eye4 = jnp.eye(4, dtype=_f32)

    p, q = _tc_pre(x, We1[:4], We1[4:], be1.reshape(1, 32), 4)
    pg, qg = _sc_gather(p, q, dst, src)
    w48 = jnp.concatenate([Wmu, Wvar, jnp.zeros((32, 28), _f32)], axis=1)
    m = _tc_edge_mlp(pg.reshape(E // 4, 128), qg.reshape(E // 4, 128),
                     jnp.kron(eye4, We2), jnp.tile(be2, 4).reshape(1, 128),
                     jnp.kron(eye4, w48), True)
    a = _sc_scatter(m.reshape(E, 32), dst, zeros_tile)

    mu, lv, p2, q2, ig = _tc_head(
        a.reshape(N, 8), eps, bmu.reshape(1, 2), bvar.reshape(1, 2),
        Wd1[:2], Wd1[2:], bd1.reshape(1, 32))

    pg2, qg2 = _sc_gather(p2, q2, dst, src)
    wd38 = jnp.concatenate([Wd3, jnp.zeros((32, 28), _f32)], axis=1)
    m2 = _tc_edge_mlp(pg2.reshape(E // 4, 128), qg2.reshape(E // 4, 128),
                      jnp.kron(eye4, Wd2), jnp.tile(bd2, 4).reshape(1, 128),
                      jnp.kron(eye4, wd38), False)
    b = _sc_scatter(m2.reshape(E, 32), dst, zeros_tile)
    out = _tc_out(b.reshape(N, 8), ig, bd3.reshape(1, 4))
    return (out, mu, lv)
